# baseline (speedup 1.0000x reference)
import jax
import jax.numpy as jnp
from jax.experimental import pallas as pl
from jax.experimental.pallas import tpu as pltpu

_L = 524288
_GRID = 2


def _balls_kernel(pos_hbm, den_ref, rad_ref, buf, sem):
    i = pl.program_id(0)
    slot = jax.lax.rem(i, 2)
    nxt = jax.lax.rem(i + 1, 2)

    @pl.when(i == 0)
    def _():
        pltpu.make_async_copy(
            pos_hbm.at[:, pl.ds(0, _L)], buf.at[0], sem.at[0]
        ).start()

    @pl.when(i + 1 < _GRID)
    def _():
        pltpu.make_async_copy(
            pos_hbm.at[:, pl.ds((i + 1) * _L, _L)], buf.at[nxt], sem.at[nxt]
        ).start()

    pltpu.make_async_copy(
        pos_hbm.at[:, pl.ds(i * _L, _L)], buf.at[slot], sem.at[slot]
    ).wait()

    x = buf[slot, 0:1, :]
    y = buf[slot, 1:2, :]
    z = buf[slot, 2:3, :]

    zz = z * z
    q1 = (jnp.square(x - 0.5) + jnp.square(y)) + zz
    q2 = (jnp.square(x + 0.5) + jnp.square(y + 0.2)) + zz
    inside = (q1 < 0.3) | (q2 < 0.8)

    m = jnp.where(inside, jnp.float32(1.0), jnp.float32(0.0))
    den_ref[...] = m
    rad_ref[0:1, :] = m
    rad_ref[1:3, :] = jnp.zeros((2, _L), jnp.float32)


@jax.jit
def _run(position):
    n = position.shape[0]
    pos_t = position.T
    den, rad = pl.pallas_call(
        _balls_kernel,
        grid=(_GRID,),
        in_specs=[pl.BlockSpec(memory_space=pl.ANY)],
        out_specs=[
            pl.BlockSpec((1, _L), lambda i: (0, i)),
            pl.BlockSpec((3, _L), lambda i: (0, i)),
        ],
        out_shape=[
            jax.ShapeDtypeStruct((1, n), jnp.float32),
            jax.ShapeDtypeStruct((3, n), jnp.float32),
        ],
        scratch_shapes=[
            pltpu.VMEM((2, 3, _L), jnp.float32),
            pltpu.SemaphoreType.DMA((2,)),
        ],
    )(pos_t)
    return den.reshape(n), rad.T


def kernel(position, direction):
    del direction
    return _run(position)
